# Initial kernel scaffold; baseline (speedup 1.0000x reference)
import functools

import jax
import jax.numpy as jnp
from jax import lax
from jax.experimental import pallas as pl
from jax.experimental.pallas import tpu as pltpu
from jax.experimental.pallas import tpu_sc as plsc

VOCAB = 100000
HID = 128
MAX_POS = 512
B, L = 1024, 200
NTOK = B * L
NLANE = 16
NVEC = HID // NLANE
NW = 32
TOK_PER_W = NTOK // NW
CHUNK = 128
NCHUNK = TOK_PER_W // CHUNK
EPS = 1e-12


def _rsqrt16(v):
    bits = lax.bitcast_convert_type(v, jnp.int32)
    y = lax.bitcast_convert_type(
        jnp.int32(0x5F3759DF) - lax.shift_right_arithmetic(bits, 1),
        jnp.float32)
    for _ in range(3):
        y = y * (1.5 - 0.5 * v * y * y)
    return y


def _make_kernel():
    mesh = plsc.VectorSubcoreMesh(core_axis_name="c", subcore_axis_name="s")

    @functools.partial(
        pl.kernel,
        mesh=mesh,
        out_type=jax.ShapeDtypeStruct((NTOK, HID), jnp.float32),
        scratch_types=[
            pltpu.VMEM((CHUNK,), jnp.int32),
            pltpu.VMEM((CHUNK, HID), jnp.float32),
            pltpu.VMEM((L, HID), jnp.float32),
            pltpu.VMEM((HID,), jnp.float32),
            pltpu.VMEM((HID,), jnp.float32),
            pltpu.SemaphoreType.DMA,
        ],
    )
    def emb_ln(ids_hbm, word_hbm, pos_hbm, gamma_hbm, beta_hbm, out_hbm,
               idx_v, rows_v, pos_v, g_v, b_v, sem):
        wid = lax.axis_index("s") * 2 + lax.axis_index("c")

        pltpu.sync_copy(pos_hbm.at[pl.ds(0, L)], pos_v)
        pltpu.sync_copy(gamma_hbm, g_v)
        pltpu.sync_copy(beta_hbm, b_v)

        g = [g_v[pl.ds(k * NLANE, NLANE)] for k in range(NVEC)]
        bt = [b_v[pl.ds(k * NLANE, NLANE)] for k in range(NVEC)]

        def row_body(r, tok_base):
            pidx = lax.rem(tok_base + r, L)
            e = []
            s = None
            s2 = None
            for k in range(NVEC):
                w = rows_v[r, pl.ds(k * NLANE, NLANE)]
                p = pos_v[pidx, pl.ds(k * NLANE, NLANE)]
                ek = w + p
                e.append(ek)
                s = ek if s is None else s + ek
                s2 = ek * ek if s2 is None else s2 + ek * ek
            mean = jnp.sum(s) * (1.0 / HID)
            var = jnp.sum(s2) * (1.0 / HID) - mean * mean
            inv = _rsqrt16(jnp.full((NLANE,), var + EPS, jnp.float32))
            mv = jnp.full((NLANE,), mean, jnp.float32)
            for k in range(NVEC):
                x = (e[k] - mv) * inv
                rows_v[r, pl.ds(k * NLANE, NLANE)] = x * g[k] + bt[k]
            return tok_base

        def chunk_body(c, _):
            tok_base = wid * TOK_PER_W + c * CHUNK
            pltpu.sync_copy(ids_hbm.at[pl.ds(tok_base, CHUNK)], idx_v)
            pltpu.async_copy(word_hbm.at[idx_v], rows_v, sem).wait()
            lax.fori_loop(0, CHUNK, row_body, tok_base)
            pltpu.sync_copy(rows_v, out_hbm.at[pl.ds(tok_base, CHUNK)])
            return 0

        lax.fori_loop(0, NCHUNK, chunk_body, 0)

    return emb_ln


_emb_ln = _make_kernel()


@jax.jit
def kernel(input_ids, word_emb, pos_emb, gamma, beta):
    ids_flat = input_ids.reshape(NTOK)
    out = _emb_ln(ids_flat, word_emb, pos_emb, gamma, beta)
    return out.reshape(B, L, HID)


# speedup vs baseline: 3.4345x; 3.4345x over previous
import functools

import jax
import jax.numpy as jnp
from jax import lax
from jax.experimental import pallas as pl
from jax.experimental.pallas import tpu as pltpu
from jax.experimental.pallas import tpu_sc as plsc

VOCAB = 100000
HID = 128
MAX_POS = 512
B, L = 1024, 200
NTOK = B * L
NLANE = 16
NVEC = HID // NLANE
NW = 32
TOK_PER_W = NTOK // NW
CHUNK = 128
NCHUNK = TOK_PER_W // CHUNK
EPS = 1e-12


_GDN = lax.GatherDimensionNumbers(
    offset_dims=(), collapsed_slice_dims=(0,), start_index_map=(0,))


def _shuffle16(v, idx):
    return lax.gather(v, idx[:, None], _GDN, slice_sizes=(1,),
                      mode=lax.GatherScatterMode.PROMISE_IN_BOUNDS)


def _hsum16(v):
    for o in (1, 2, 4, 8):
        idx = lax.iota(jnp.int32, 16) ^ o
        v = v + _shuffle16(v, idx)
    return v


def _rsqrt16(v):
    bits = lax.bitcast_convert_type(v, jnp.int32)
    y = lax.bitcast_convert_type(
        jnp.int32(0x5F3759DF) - lax.shift_right_arithmetic(bits, 1),
        jnp.float32)
    for _ in range(3):
        y = y * (1.5 - 0.5 * v * y * y)
    return y


def _make_kernel():
    mesh = plsc.VectorSubcoreMesh(core_axis_name="c", subcore_axis_name="s")

    @functools.partial(
        pl.kernel,
        mesh=mesh,
        out_type=jax.ShapeDtypeStruct((NTOK, HID), jnp.float32),
        scratch_types=[
            pltpu.VMEM((CHUNK,), jnp.int32),
            pltpu.VMEM((CHUNK, HID), jnp.float32),
            pltpu.VMEM((L, HID), jnp.float32),
            pltpu.VMEM((HID,), jnp.float32),
            pltpu.VMEM((HID,), jnp.float32),
            pltpu.SemaphoreType.DMA,
        ],
    )
    def emb_ln(ids_hbm, word_hbm, pos_hbm, gamma_hbm, beta_hbm, out_hbm,
               idx_v, rows_v, pos_v, g_v, b_v, sem):
        wid = lax.axis_index("s") * 2 + lax.axis_index("c")

        pltpu.sync_copy(pos_hbm.at[pl.ds(0, L)], pos_v)
        pltpu.sync_copy(gamma_hbm, g_v)
        pltpu.sync_copy(beta_hbm, b_v)

        g = [g_v[pl.ds(k * NLANE, NLANE)] for k in range(NVEC)]
        bt = [b_v[pl.ds(k * NLANE, NLANE)] for k in range(NVEC)]

        def row_body(r, tok_base):
            pidx = lax.rem(tok_base + r, L)
            e = []
            s = None
            s2 = None
            for k in range(NVEC):
                w = rows_v[r, pl.ds(k * NLANE, NLANE)]
                p = pos_v[pidx, pl.ds(k * NLANE, NLANE)]
                ek = w + p
                e.append(ek)
                s = ek if s is None else s + ek
                s2 = ek * ek if s2 is None else s2 + ek * ek
            mv = _hsum16(s) * (1.0 / HID)
            var = _hsum16(s2) * (1.0 / HID) - mv * mv
            inv = _rsqrt16(var + EPS)
            for k in range(NVEC):
                x = (e[k] - mv) * inv
                rows_v[r, pl.ds(k * NLANE, NLANE)] = x * g[k] + bt[k]
            return tok_base

        def chunk_body(c, _):
            tok_base = wid * TOK_PER_W + c * CHUNK
            pltpu.sync_copy(ids_hbm.at[pl.ds(tok_base, CHUNK)], idx_v)
            pltpu.async_copy(word_hbm.at[idx_v], rows_v, sem).wait()
            lax.fori_loop(0, CHUNK, row_body, tok_base)
            pltpu.sync_copy(rows_v, out_hbm.at[pl.ds(tok_base, CHUNK)])
            return 0

        lax.fori_loop(0, NCHUNK, chunk_body, 0)

    return emb_ln


_emb_ln = _make_kernel()


@jax.jit
def kernel(input_ids, word_emb, pos_emb, gamma, beta):
    ids_flat = input_ids.reshape(NTOK)
    out = _emb_ln(ids_flat, word_emb, pos_emb, gamma, beta)
    return out.reshape(B, L, HID)
